# Initial kernel scaffold; baseline (speedup 1.0000x reference)
import jax
import jax.numpy as jnp
from jax.experimental import pallas as pl


def kernel(sequence, token_table, pe):
    raise NotImplementedError("write your pallas kernel here")


# speedup vs baseline: 2.7279x; 2.7279x over previous
import functools

import jax
import jax.numpy as jnp
from jax import lax
from jax.experimental import pallas as pl
from jax.experimental.pallas import tpu as pltpu
from jax.experimental.pallas import tpu_sc as plsc

B = 4096
L = 200
D = 128
NW = 32
ROWS_PER_W = B // NW


@functools.partial(
    pl.kernel,
    out_type=jax.ShapeDtypeStruct((B, L, D), jnp.float32),
    mesh=plsc.VectorSubcoreMesh(core_axis_name="c", subcore_axis_name="s"),
    scratch_types=[
        pltpu.VMEM((L,), jnp.int32),
        pltpu.VMEM((L, D), jnp.float32),
        pltpu.VMEM((L, D), jnp.float32),
        pltpu.SemaphoreType.DMA,
    ],
)
def _embed_sc(seq_hbm, table_hbm, pe_hbm, out_hbm, idx_v, pe_v, rows_v, sem):
    wid = lax.axis_index("s") * 2 + lax.axis_index("c")
    base_row = wid * ROWS_PER_W

    pltpu.sync_copy(pe_hbm, pe_v)

    def per_row(r, carry):
        row = base_row + r
        tok_base = row * L
        pltpu.sync_copy(seq_hbm.at[pl.ds(tok_base, L)], idx_v)
        cp0 = pltpu.async_copy(
            table_hbm.at[idx_v.at[pl.ds(0, 120)]], rows_v.at[pl.ds(0, 120)], sem
        )
        cp1 = pltpu.async_copy(
            table_hbm.at[idx_v.at[pl.ds(120, 80)]], rows_v.at[pl.ds(120, 80)], sem
        )
        cp0.wait()
        cp1.wait()

        def add_pe(l, c):
            for j in range(D // 16):
                sl = pl.ds(j * 16, 16)
                rows_v[l, sl] = rows_v[l, sl] + pe_v[l, sl]
            return c

        lax.fori_loop(0, L, add_pe, 0)
        pltpu.sync_copy(rows_v, out_hbm.at[row])
        return carry

    lax.fori_loop(0, ROWS_PER_W, per_row, 0)


def kernel(sequence, token_table, pe):
    seq_flat = sequence.reshape(B * L).astype(jnp.int32)
    pe_slice = pe[0, :L, :]
    return _embed_sc(seq_flat, token_table, pe_slice)


# speedup vs baseline: 6.8499x; 2.5111x over previous
import functools

import jax
import jax.numpy as jnp
from jax import lax
from jax.experimental import pallas as pl
from jax.experimental.pallas import tpu as pltpu
from jax.experimental.pallas import tpu_sc as plsc

B = 4096
L = 200
D = 128
V = 129
VP = 136
NW = 32
ROWS_PER_W = B // NW
C0, C1 = 120, 80


def _build_body(table_ref, pe_ref, out_ref):
    out_ref[...] = table_ref[...] + pe_ref[0]


_build_combined = pl.pallas_call(
    _build_body,
    grid=(L,),
    in_specs=[
        pl.BlockSpec((VP, D), lambda l: (0, 0)),
        pl.BlockSpec((1, 1, D), lambda l: (l, 0, 0)),
    ],
    out_specs=pl.BlockSpec((VP, D), lambda l: (l, 0)),
    out_shape=jax.ShapeDtypeStruct((L * VP, D), jnp.float32),
)


@functools.partial(
    pl.kernel,
    out_type=jax.ShapeDtypeStruct((B, L, D), jnp.float32),
    mesh=plsc.VectorSubcoreMesh(core_axis_name="c", subcore_axis_name="s"),
    scratch_types=[
        pltpu.VMEM((ROWS_PER_W * L,), jnp.int32),
        pltpu.VMEM((L,), jnp.int32),
        pltpu.VMEM((16,), jnp.int32),
        pltpu.VMEM((L, D), jnp.float32),
        pltpu.VMEM((L, D), jnp.float32),
        pltpu.SemaphoreType.DMA,
        pltpu.SemaphoreType.DMA,
        pltpu.SemaphoreType.DMA,
        pltpu.SemaphoreType.DMA,
    ],
)
def _embed_sc(seq_hbm, comb_hbm, offs_hbm, offst_hbm, out_hbm,
              idx_all, offs_v, offst_v, buf_a, buf_b,
              gsem_a, gsem_b, ssem_a, ssem_b):
    wid = lax.axis_index("s") * 2 + lax.axis_index("c")
    base_row = wid * ROWS_PER_W

    pltpu.sync_copy(
        seq_hbm.at[pl.ds(base_row * L, ROWS_PER_W * L)], idx_all
    )
    pltpu.sync_copy(offs_hbm, offs_v)
    pltpu.sync_copy(offst_hbm, offst_v)

    def fix_row(r, c):
        rb = r * L
        for k in range(12):
            sl = pl.ds(rb + k * 16, 16)
            idx_all[sl] = idx_all[sl] + offs_v[pl.ds(k * 16, 16)]
        sl = pl.ds(rb + 184, 16)
        idx_all[sl] = idx_all[sl] + offst_v[...]
        return c

    lax.fori_loop(0, ROWS_PER_W, fix_row, 0)

    def start_gather(r, buf, sem):
        rb = r * L
        pltpu.async_copy(
            comb_hbm.at[idx_all.at[pl.ds(rb, C0)]], buf.at[pl.ds(0, C0)], sem
        )
        pltpu.async_copy(
            comb_hbm.at[idx_all.at[pl.ds(rb + C0, C1)]], buf.at[pl.ds(C0, C1)], sem
        )

    def wait_gather(r, buf, sem):
        rb = r * L
        pltpu.make_async_copy(
            comb_hbm.at[idx_all.at[pl.ds(rb, C0)]], buf.at[pl.ds(0, C0)], sem
        ).wait()
        pltpu.make_async_copy(
            comb_hbm.at[idx_all.at[pl.ds(rb + C0, C1)]], buf.at[pl.ds(C0, C1)], sem
        ).wait()

    def start_store(row, buf, sem):
        pltpu.async_copy(buf, out_hbm.at[row], sem)

    def wait_store(row, buf, sem):
        pltpu.make_async_copy(buf, out_hbm.at[row], sem).wait()

    start_gather(0, buf_a, gsem_a)
    start_gather(1, buf_b, gsem_b)

    def pair(g, c):
        r0 = 2 * g
        r1 = r0 + 1
        wait_gather(r0, buf_a, gsem_a)
        start_store(base_row + r0, buf_a, ssem_a)
        wait_gather(r1, buf_b, gsem_b)
        start_store(base_row + r1, buf_b, ssem_b)

        @pl.when(g < ROWS_PER_W // 2 - 1)
        def _prefetch():
            wait_store(base_row + r0, buf_a, ssem_a)
            start_gather(r0 + 2, buf_a, gsem_a)
            wait_store(base_row + r1, buf_b, ssem_b)
            start_gather(r1 + 2, buf_b, gsem_b)

        return c

    lax.fori_loop(0, ROWS_PER_W // 2, pair, 0)
    wait_store(base_row, buf_a, ssem_a)
    wait_store(base_row, buf_b, ssem_b)


def kernel(sequence, token_table, pe):
    pe_slice = pe[0, :L, :].reshape(L, 1, D)
    table_pad = jnp.pad(token_table, ((0, VP - V), (0, 0)))
    combined = _build_combined(table_pad, pe_slice)
    seq = sequence.reshape(B * L).astype(jnp.int32)
    offs = jnp.arange(L, dtype=jnp.int32) * VP
    offs_tail = jnp.concatenate(
        [jnp.zeros(8, jnp.int32), jnp.arange(192, 200, dtype=jnp.int32) * VP]
    )
    return _embed_sc(seq, combined, offs, offs_tail)


# speedup vs baseline: 8.4119x; 1.2280x over previous
import functools

import jax
import jax.numpy as jnp
from jax import lax
from jax.experimental import pallas as pl
from jax.experimental.pallas import tpu as pltpu
from jax.experimental.pallas import tpu_sc as plsc

B = 4096
L = 200
D = 128
V = 129
VP = 136
NW = 32
ROWS_PER_W = B // NW
TOK_PER_W = ROWS_PER_W * L
CH = 128
NCHUNK = TOK_PER_W // CH
NBUF = 4
NGROUP = NCHUNK // NBUF
LB = 8


def _build_body(table_ref, pe_ref, out_ref):
    t = table_ref[...]
    for j in range(LB):
        out_ref[pl.ds(j * VP, VP)] = t + pe_ref[j]


_build_combined = pl.pallas_call(
    _build_body,
    grid=(L // LB,),
    in_specs=[
        pl.BlockSpec((VP, D), lambda i: (0, 0)),
        pl.BlockSpec((LB, 1, D), lambda i: (i, 0, 0)),
    ],
    out_specs=pl.BlockSpec((LB * VP, D), lambda i: (i, 0)),
    out_shape=jax.ShapeDtypeStruct((L * VP, D), jnp.float32),
)


@functools.partial(
    pl.kernel,
    out_type=jax.ShapeDtypeStruct((B * L, D), jnp.float32),
    mesh=plsc.VectorSubcoreMesh(core_axis_name="c", subcore_axis_name="s"),
    scratch_types=[
        pltpu.VMEM((TOK_PER_W,), jnp.int32),
        pltpu.VMEM((L,), jnp.int32),
        pltpu.VMEM((16,), jnp.int32),
        pltpu.VMEM((CH, D), jnp.float32),
        pltpu.VMEM((CH, D), jnp.float32),
        pltpu.VMEM((CH, D), jnp.float32),
        pltpu.VMEM((CH, D), jnp.float32),
        pltpu.SemaphoreType.DMA,
        pltpu.SemaphoreType.DMA,
        pltpu.SemaphoreType.DMA,
        pltpu.SemaphoreType.DMA,
        pltpu.SemaphoreType.DMA,
        pltpu.SemaphoreType.DMA,
        pltpu.SemaphoreType.DMA,
        pltpu.SemaphoreType.DMA,
    ],
)
def _embed_sc(seq_hbm, comb_hbm, offs_hbm, offst_hbm, out_hbm,
              idx_all, offs_v, offst_v, buf0, buf1, buf2, buf3,
              gsem0, gsem1, gsem2, gsem3, ssem0, ssem1, ssem2, ssem3):
    wid = lax.axis_index("s") * 2 + lax.axis_index("c")
    base_tok = wid * TOK_PER_W
    bufs = (buf0, buf1, buf2, buf3)
    gsems = (gsem0, gsem1, gsem2, gsem3)
    ssems = (ssem0, ssem1, ssem2, ssem3)

    pltpu.sync_copy(seq_hbm.at[pl.ds(base_tok, TOK_PER_W)], idx_all)
    pltpu.sync_copy(offs_hbm, offs_v)
    pltpu.sync_copy(offst_hbm, offst_v)

    def fix_row(r, c):
        rb = r * L
        for k in range(12):
            sl = pl.ds(rb + k * 16, 16)
            idx_all[sl] = idx_all[sl] + offs_v[pl.ds(k * 16, 16)]
        sl = pl.ds(rb + 184, 16)
        idx_all[sl] = idx_all[sl] + offst_v[...]
        return c

    lax.fori_loop(0, ROWS_PER_W, fix_row, 0)

    def start_g(c, b):
        pltpu.async_copy(
            comb_hbm.at[idx_all.at[pl.ds(c * CH, CH)]], bufs[b], gsems[b]
        )

    def wait_g(c, b):
        pltpu.make_async_copy(
            comb_hbm.at[idx_all.at[pl.ds(c * CH, CH)]], bufs[b], gsems[b]
        ).wait()

    def start_s(c, b):
        pltpu.async_copy(
            bufs[b], out_hbm.at[pl.ds(base_tok + c * CH, CH)], ssems[b]
        )

    def wait_s(c, b):
        pltpu.make_async_copy(
            bufs[b], out_hbm.at[pl.ds(base_tok + c * CH, CH)], ssems[b]
        ).wait()

    for b in range(NBUF):
        start_g(b, b)

    def group(g, carry):
        c0 = NBUF * g
        for b in range(NBUF):
            wait_g(c0 + b, b)
            start_s(c0 + b, b)

        @pl.when(g < NGROUP - 1)
        def _prefetch():
            for b in range(NBUF):
                wait_s(c0 + b, b)
                start_g(c0 + NBUF + b, b)

        return carry

    lax.fori_loop(0, NGROUP, group, 0)
    for b in range(NBUF):
        wait_s(0, b)


def kernel(sequence, token_table, pe):
    pe_slice = pe[0, :L, :].reshape(L, 1, D)
    table_pad = jnp.pad(token_table, ((0, VP - V), (0, 0)))
    combined = _build_combined(table_pad, pe_slice)
    seq = sequence.reshape(B * L).astype(jnp.int32)
    offs = jnp.arange(L, dtype=jnp.int32) * VP
    offs_tail = jnp.concatenate(
        [jnp.zeros(8, jnp.int32), jnp.arange(192, 200, dtype=jnp.int32) * VP]
    )
    out = _embed_sc(seq, combined, offs, offs_tail)
    return out.reshape(B, L, D)
